# baseline (speedup 1.0000x reference)
import functools

import jax
import jax.numpy as jnp
from jax.experimental import pallas as pl
from jax.experimental.pallas import tpu as pltpu


def _single_buffered(block_shape, index_map):
    try:
        return pl.BlockSpec(block_shape, index_map,
                            pipeline_mode=pl.Buffered(buffer_count=1))
    except Exception:
        return pl.BlockSpec(block_shape, index_map)


def _fused_kernel(x_ref, w_ref, g_ref, b_ref, o_ref, s_ref, *,
                  ho, wo, h, w, cin, groups, eps, slope):
    hw = ho * wo
    cout = o_ref.shape[2]

    zc = jnp.zeros((h + 4, 8, cin), jnp.float32)
    s_ref[:, 0:8, :] = zc
    s_ref[:, w + 8:w + 16, :] = zc
    zr = jnp.zeros((2, w, cin), jnp.float32)
    s_ref[0:2, 8:w + 8, :] = zr
    s_ref[h + 2:h + 4, 8:w + 8, :] = zr

    s_ref[2:h + 2, 8:w + 8, :] = x_ref[0].T.reshape(h, w, cin)

    acc = jnp.zeros((hw, cout), jnp.float32)
    for py in range(2):
        for dx in range(2):
            for px in range(2):
                a = s_ref[pl.ds(py + 1, ho + 1, 2),
                          pl.ds(2 * dx + px + 7, wo, 2), :]
                a = a.reshape((ho + 1) * wo, cin).astype(jnp.bfloat16)
                for dy in range(2):
                    t = 8 * dy + 4 * dx + 2 * py + px
                    acc += jnp.dot(a[wo * dy:wo * dy + hw], w_ref[t],
                                   preferred_element_type=jnp.float32)

    cg = cout // groups
    s1 = jnp.sum(acc, axis=0, keepdims=True)
    s2 = jnp.sum(acc * acc, axis=0, keepdims=True)
    li = jax.lax.broadcasted_iota(jnp.int32, (cout, cout), 0) // cg
    lj = jax.lax.broadcasted_iota(jnp.int32, (cout, cout), 1) // cg
    agg = (li == lj).astype(jnp.float32)
    n = float(hw * cg)
    mean = jax.lax.dot(s1, agg,
                       precision=jax.lax.Precision.HIGHEST) / n
    ex2 = jax.lax.dot(s2, agg,
                      precision=jax.lax.Precision.HIGHEST) / n
    var = jnp.maximum(ex2 - mean * mean, 0.0)
    inv = jax.lax.rsqrt(var + eps)
    scale = inv * g_ref[...]
    bias = b_ref[...] - mean * scale

    z = acc * scale + bias
    o_ref[0] = jnp.where(z >= 0.0, z, slope * z).astype(o_ref.dtype)


def _transpose_kernel(z_ref, o_ref):
    o_ref[0] = z_ref[0].T


def kernel(x_nchw, w_hwio, gamma, beta, *, num_groups=4, eps=1e-5,
           negative_slope=0.2):
    B, Cin, H, W = x_nchw.shape
    KH, KW, wcin, Cout = w_hwio.shape
    assert (KH, KW) == (4, 4) and wcin == Cin and H % 8 == 0 and W % 8 == 0
    Ho, Wo = H // 2, W // 2
    HW = Ho * Wo
    in_dtype = x_nchw.dtype

    xf = x_nchw.reshape(B, Cin, H * W)

    w16 = (w_hwio.reshape(2, 2, 2, 2, Cin, Cout)
                 .transpose(0, 2, 1, 3, 4, 5)
                 .reshape(16, Cin, Cout)
                 .astype(jnp.bfloat16))
    g2 = gamma.reshape(1, Cout).astype(jnp.float32)
    b2 = beta.reshape(1, Cout).astype(jnp.float32)

    cparams = pltpu.CompilerParams(
        dimension_semantics=("parallel",),
        vmem_limit_bytes=48 * 1024 * 1024)

    out = pl.pallas_call(
        functools.partial(_fused_kernel, ho=Ho, wo=Wo, h=H, w=W, cin=Cin,
                          groups=num_groups, eps=eps, slope=negative_slope),
        grid=(B,),
        in_specs=[
            pl.BlockSpec((1, Cin, H * W), lambda b: (b, 0, 0)),
            _single_buffered((16, Cin, Cout), lambda b: (0, 0, 0)),
            _single_buffered((1, Cout), lambda b: (0, 0)),
            _single_buffered((1, Cout), lambda b: (0, 0)),
        ],
        out_specs=pl.BlockSpec((1, HW, Cout), lambda b: (b, 0, 0)),
        out_shape=jax.ShapeDtypeStruct((B, HW, Cout), in_dtype),
        scratch_shapes=[pltpu.VMEM((H + 4, W + 16, Cin), jnp.float32)],
        compiler_params=cparams,
    )(xf, w16, g2, b2)

    out = pl.pallas_call(
        _transpose_kernel,
        grid=(B,),
        in_specs=[pl.BlockSpec((1, HW, Cout), lambda b: (b, 0, 0))],
        out_specs=pl.BlockSpec((1, Cout, HW), lambda b: (b, 0, 0)),
        out_shape=jax.ShapeDtypeStruct((B, Cout, HW), in_dtype),
        compiler_params=cparams,
    )(out)

    return out.reshape(B, Cout, Ho, Wo)


# speedup vs baseline: 1.3294x; 1.3294x over previous
import functools

import jax
import jax.numpy as jnp
from jax.experimental import pallas as pl
from jax.experimental.pallas import tpu as pltpu


def _single_buffered(block_shape, index_map):
    try:
        return pl.BlockSpec(block_shape, index_map,
                            pipeline_mode=pl.Buffered(buffer_count=1))
    except Exception:
        return pl.BlockSpec(block_shape, index_map)


def _fused_kernel(x_ref, w_ref, g_ref, b_ref, o_ref, s_ref, *,
                  ho, wo, h, w, cin, groups, eps, slope):
    hw = ho * wo
    cout = o_ref.shape[2]

    zc = jnp.zeros((h + 4, 8, cin), jnp.float32)
    s_ref[:, 0:8, :] = zc
    s_ref[:, w + 8:w + 16, :] = zc
    zr = jnp.zeros((2, w, cin), jnp.float32)
    s_ref[0:2, 8:w + 8, :] = zr
    s_ref[h + 2:h + 4, 8:w + 8, :] = zr

    s_ref[2:h + 2, 8:w + 8, :] = x_ref[0].T.reshape(h, w, cin)

    acc = jnp.zeros((hw, cout), jnp.float32)
    for py in range(2):
        for dx in range(2):
            for px in range(2):
                a = s_ref[pl.ds(py + 1, ho + 1, 2),
                          pl.ds(2 * dx + px + 7, wo, 2), :]
                a = a.reshape((ho + 1) * wo, cin).astype(jnp.bfloat16)
                for dy in range(2):
                    t = 8 * dy + 4 * dx + 2 * py + px
                    acc += jnp.dot(a[wo * dy:wo * dy + hw], w_ref[t],
                                   preferred_element_type=jnp.float32)

    cg = cout // groups
    s1 = jnp.sum(acc, axis=0, keepdims=True)
    s2 = jnp.sum(acc * acc, axis=0, keepdims=True)
    li = jax.lax.broadcasted_iota(jnp.int32, (cout, cout), 0) // cg
    lj = jax.lax.broadcasted_iota(jnp.int32, (cout, cout), 1) // cg
    agg = (li == lj).astype(jnp.float32)
    n = float(hw * cg)
    mean = jax.lax.dot(s1, agg,
                       precision=jax.lax.Precision.HIGHEST) / n
    ex2 = jax.lax.dot(s2, agg,
                      precision=jax.lax.Precision.HIGHEST) / n
    var = jnp.maximum(ex2 - mean * mean, 0.0)
    inv = jax.lax.rsqrt(var + eps)
    scale = inv * g_ref[...]
    bias = b_ref[...] - mean * scale

    z = acc * scale + bias
    o_ref[0] = jnp.where(z >= 0.0, z, slope * z).astype(o_ref.dtype)


def kernel(x_nchw, w_hwio, gamma, beta, *, num_groups=4, eps=1e-5,
           negative_slope=0.2):
    B, Cin, H, W = x_nchw.shape
    KH, KW, wcin, Cout = w_hwio.shape
    assert (KH, KW) == (4, 4) and wcin == Cin and H % 8 == 0 and W % 8 == 0
    Ho, Wo = H // 2, W // 2
    HW = Ho * Wo
    in_dtype = x_nchw.dtype

    xf = x_nchw.reshape(B, Cin, H * W)

    w16 = (w_hwio.reshape(2, 2, 2, 2, Cin, Cout)
                 .transpose(0, 2, 1, 3, 4, 5)
                 .reshape(16, Cin, Cout)
                 .astype(jnp.bfloat16))
    g2 = gamma.reshape(1, Cout).astype(jnp.float32)
    b2 = beta.reshape(1, Cout).astype(jnp.float32)

    cparams = pltpu.CompilerParams(
        dimension_semantics=("parallel",),
        vmem_limit_bytes=48 * 1024 * 1024)

    out = pl.pallas_call(
        functools.partial(_fused_kernel, ho=Ho, wo=Wo, h=H, w=W, cin=Cin,
                          groups=num_groups, eps=eps, slope=negative_slope),
        grid=(B,),
        in_specs=[
            pl.BlockSpec((1, Cin, H * W), lambda b: (b, 0, 0)),
            _single_buffered((16, Cin, Cout), lambda b: (0, 0, 0)),
            _single_buffered((1, Cout), lambda b: (0, 0)),
            _single_buffered((1, Cout), lambda b: (0, 0)),
        ],
        out_specs=pl.BlockSpec((1, HW, Cout), lambda b: (b, 0, 0)),
        out_shape=jax.ShapeDtypeStruct((B, HW, Cout), in_dtype),
        scratch_shapes=[pltpu.VMEM((H + 4, W + 16, Cin), jnp.float32)],
        compiler_params=cparams,
    )(xf, w16, g2, b2)

    return jnp.transpose(out.reshape(B, Ho, Wo, Cout), (0, 3, 1, 2))
